# Initial kernel scaffold; baseline (speedup 1.0000x reference)
import jax
import jax.numpy as jnp
from jax.experimental import pallas as pl


def kernel(z, W):
    raise NotImplementedError("write your pallas kernel here")


# speedup vs baseline: 1.2231x; 1.2231x over previous
import functools

import jax
import jax.numpy as jnp
from jax import lax
from jax.experimental import pallas as pl
from jax.experimental.pallas import tpu as pltpu

_V = 8192
_D = 32
_BETA = 0.25
_ROWS = 256


def _argmin_body(z_ref, w_ref, idx_ref):
    zt = z_ref[...]
    w = w_ref[...]
    t = lax.dot_general(zt, w, (((1,), (1,)), ((), ())),
                        preferred_element_type=jnp.float32)
    zsq = jnp.sum(zt * zt, axis=1, keepdims=True)
    wsq = jnp.sum(w * w, axis=1)
    d = (zsq + wsq[None, :]) - 2.0 * t
    minval = jnp.min(d, axis=1, keepdims=True)
    js = lax.broadcasted_iota(jnp.int32, d.shape, 1)
    idx = jnp.min(jnp.where(d == minval, js, _V), axis=1)
    idx_ref[0, 0, :] = idx.astype(jnp.int32)


def _argmin_call(z_flat, W):
    nblocks = z_flat.shape[0] // _ROWS
    out = pl.pallas_call(
        _argmin_body,
        grid=(nblocks,),
        in_specs=[pl.BlockSpec((_ROWS, _D), lambda i: (i, 0)),
                  pl.BlockSpec((_V, _D), lambda i: (0, 0))],
        out_specs=pl.BlockSpec((1, 1, _ROWS), lambda i: (i, 0, 0)),
        out_shape=jax.ShapeDtypeStruct((nblocks, 1, _ROWS), jnp.int32),
    )(z_flat, W)
    return out.reshape(-1)


def kernel(z, W):
    zp = jnp.transpose(z, (0, 2, 3, 1))
    z_flat = zp.reshape(-1, _D)
    n = z_flat.shape[0]
    idx = _argmin_call(z_flat, W)

    zq_flat = W[idx]
    zq = z_flat + (zq_flat - z_flat)
    m1 = jnp.mean((zq_flat - z_flat) ** 2)
    loss = m1 + _BETA * m1
    counts = jnp.zeros((_V,), jnp.float32).at[idx].add(1.0)
    e = counts / n
    perp = jnp.exp(-jnp.sum(e * jnp.log(e + 1e-10)))
    mean_d = (jnp.mean(jnp.sum(z_flat * z_flat, axis=1))
              + jnp.mean(jnp.sum(W * W, axis=1))
              - 2.0 * jnp.dot(jnp.mean(z_flat, axis=0), jnp.mean(W, axis=0)))
    z_q_out = jnp.transpose(zq.reshape(zp.shape), (0, 3, 1, 2))
    return (z_q_out, loss, perp, idx, mean_d)


# speedup vs baseline: 1.7816x; 1.4566x over previous
import functools

import jax
import jax.numpy as jnp
from jax import lax
from jax.experimental import pallas as pl
from jax.experimental.pallas import tpu as pltpu
from jax.experimental.pallas import tpu_sc as plsc

_V = 8192
_D = 32
_B = 4096
_BETA = 0.25
_ROWS = 256

_NC = 2
_NS = 16
_NW = _NC * _NS
_BPW = _B // _NW


def _argmin_body(z_ref, w_ref, idx_ref):
    zt = z_ref[...]
    w = w_ref[...]
    t = lax.dot_general(zt, w, (((1,), (1,)), ((), ())),
                        preferred_element_type=jnp.float32)
    zsq = jnp.sum(zt * zt, axis=1, keepdims=True)
    wsq = jnp.sum(w * w, axis=1)
    d = (zsq + wsq[None, :]) - 2.0 * t
    minval = jnp.min(d, axis=1, keepdims=True)
    js = lax.broadcasted_iota(jnp.int32, d.shape, 1)
    idx = jnp.min(jnp.where(d == minval, js, _V), axis=1)
    idx_ref[0, 0, :] = idx.astype(jnp.int32)


def _argmin_call(z_flat, W):
    nblocks = _B // _ROWS
    out = pl.pallas_call(
        _argmin_body,
        grid=(nblocks,),
        in_specs=[pl.BlockSpec((_ROWS, _D), lambda i: (i, 0)),
                  pl.BlockSpec((_V, _D), lambda i: (0, 0))],
        out_specs=pl.BlockSpec((1, 1, _ROWS), lambda i: (i, 0, 0)),
        out_shape=jax.ShapeDtypeStruct((nblocks, 1, _ROWS), jnp.int32),
    )(z_flat, W)
    return out.reshape(-1)


def _sc_body(w_hbm, idx_hbm, z_hbm, ones_hbm, zeros_hbm,
             zq_hbm, counts_hbm,
             idx_v, rows_v, z_v, ones_v, hist_sh, sem):
    cid = lax.axis_index("c")
    sid = lax.axis_index("s")
    wid = sid * _NC + cid
    base = wid * _BPW

    pltpu.sync_copy(idx_hbm.at[pl.ds(base, _BPW)], idx_v)
    gather = pltpu.async_copy(w_hbm.at[idx_v], rows_v, sem)
    pltpu.sync_copy(z_hbm.at[pl.ds(base, _BPW)], z_v)
    pltpu.sync_copy(ones_hbm, ones_v)

    @pl.when(sid == 0)
    def _():
        pltpu.sync_copy(zeros_hbm, hist_sh)

    gather.wait()
    plsc.subcore_barrier()

    pltpu.sync_copy(ones_v, hist_sh.at[idx_v], add=True)

    def _rows(r, carry):
        for h in range(_D // 16):
            zv = z_v[r, pl.ds(h * 16, 16)]
            wv = rows_v[r, pl.ds(h * 16, 16)]
            z_v[r, pl.ds(h * 16, 16)] = zv + (wv - zv)
        return carry
    lax.fori_loop(0, _BPW, _rows, 0)

    pltpu.sync_copy(z_v, zq_hbm.at[pl.ds(base, _BPW)])

    plsc.subcore_barrier()

    @pl.when(sid == 0)
    def _():
        pltpu.sync_copy(hist_sh, counts_hbm.at[cid])


def _sc_call(W, idx, z_flat, ones, zeros):
    mesh = plsc.VectorSubcoreMesh(core_axis_name="c", subcore_axis_name="s")
    f = pl.kernel(
        _sc_body,
        out_type=[jax.ShapeDtypeStruct((_B, _D), jnp.float32),
                  jax.ShapeDtypeStruct((_NC, _V), jnp.float32)],
        mesh=mesh,
        scratch_types=[
            pltpu.VMEM((_BPW,), jnp.int32),
            pltpu.VMEM((_BPW, 128), jnp.float32),
            pltpu.VMEM((_BPW, _D), jnp.float32),
            pltpu.VMEM((_BPW,), jnp.float32),
            pltpu.VMEM_SHARED((_V,), jnp.float32),
            pltpu.SemaphoreType.DMA,
        ],
    )
    return f(W, idx, z_flat, ones, zeros)


def _stats_body(z_ref, w_ref, zq_ref, c_ref, loss_ref, perp_ref, md_ref):
    z = z_ref[...]
    w = w_ref[...]
    q = zq_ref[...]
    diff = q - z
    m = jnp.sum(diff * diff) * (1.0 / (_B * _D))
    loss_ref[...] = (m + _BETA * m).reshape(1, 1)

    szsq = jnp.sum(z * z)
    swsq = jnp.sum(w * w)
    sz = jnp.sum(z, axis=0)
    sw = jnp.sum(w, axis=0)
    md_ref[...] = (szsq * (1.0 / _B) + swsq * (1.0 / _V)
                   - 2.0 * jnp.sum(sz * sw) * (1.0 / (_B * _V))).reshape(1, 1)

    cs = c_ref[0:1, :] + c_ref[1:2, :]
    e = cs * (1.0 / _B)
    ent = jnp.sum(e * jnp.log(e + 1e-10))
    perp_ref[...] = jnp.exp(-ent).reshape(1, 1)


def _stats_call(z_flat, W, zq, counts):
    outs = pl.pallas_call(
        _stats_body,
        out_shape=[jax.ShapeDtypeStruct((1, 1), jnp.float32),
                   jax.ShapeDtypeStruct((1, 1), jnp.float32),
                   jax.ShapeDtypeStruct((1, 1), jnp.float32)],
    )(z_flat, W, zq, counts)
    return outs[0][0, 0], outs[1][0, 0], outs[2][0, 0]


def kernel(z, W):
    zp = jnp.transpose(z, (0, 2, 3, 1))
    z_flat = zp.reshape(_B, _D)
    idx = _argmin_call(z_flat, W)

    ones = jnp.ones((_BPW,), jnp.float32)
    zeros = jnp.zeros((_V,), jnp.float32)
    W_pad = jnp.pad(W, ((0, 0), (0, 128 - _D)))
    zq_flat, counts = _sc_call(W_pad, idx, z_flat, ones, zeros)

    loss, perp, mean_d = _stats_call(z_flat, W, zq_flat, counts)

    z_q_out = jnp.transpose(zq_flat.reshape(zp.shape), (0, 3, 1, 2))
    return (z_q_out, loss, perp, idx, mean_d)


# speedup vs baseline: 1.8703x; 1.0498x over previous
import functools

import jax
import jax.numpy as jnp
from jax import lax
from jax.experimental import pallas as pl
from jax.experimental.pallas import tpu as pltpu
from jax.experimental.pallas import tpu_sc as plsc

_V = 8192
_D = 32
_B = 4096
_BETA = 0.25
_ROWS = 256

_NC = 2
_NS = 16
_NW = _NC * _NS
_BPW = _B // _NW


def _argmin_body(z_ref, w_ref, idx_ref, wpad_ref):
    zt = z_ref[...]
    w = w_ref[...]
    t2 = lax.dot_general(-2.0 * zt, w, (((1,), (1,)), ((), ())),
                         preferred_element_type=jnp.float32)
    zsq = jnp.sum(zt * zt, axis=1, keepdims=True)
    wsq = jnp.sum(w * w, axis=1)
    d = (zsq + wsq[None, :]) + t2
    minval = jnp.min(d, axis=1, keepdims=True)
    js = lax.broadcasted_iota(jnp.int32, d.shape, 1)
    idx = jnp.min(jnp.where(d == minval, js, _V), axis=1)
    idx_ref[0, 0, :] = idx.astype(jnp.int32)

    @pl.when(pl.program_id(0) == 0)
    def _():
        wpad_ref[:, 0:_D] = w


def _argmin_call(z_flat, W):
    nblocks = _B // _ROWS
    out, wpad = pl.pallas_call(
        _argmin_body,
        grid=(nblocks,),
        in_specs=[pl.BlockSpec((_ROWS, _D), lambda i: (i, 0)),
                  pl.BlockSpec((_V, _D), lambda i: (0, 0))],
        out_specs=[pl.BlockSpec((1, 1, _ROWS), lambda i: (i, 0, 0)),
                   pl.BlockSpec((_V, 128), lambda i: (0, 0))],
        out_shape=[jax.ShapeDtypeStruct((nblocks, 1, _ROWS), jnp.int32),
                   jax.ShapeDtypeStruct((_V, 128), jnp.float32)],
    )(z_flat, W)
    return out.reshape(-1), wpad


def _sc_body(w_hbm, idx_hbm, z_hbm, ones_hbm, zeros_hbm,
             zq_hbm, counts_hbm,
             idx_v, rows_v, z_v, ones_v, hist_sh, sem):
    cid = lax.axis_index("c")
    sid = lax.axis_index("s")
    wid = sid * _NC + cid
    base = wid * _BPW

    pltpu.sync_copy(idx_hbm.at[pl.ds(base, _BPW)], idx_v)
    gather = pltpu.async_copy(w_hbm.at[idx_v], rows_v, sem)
    pltpu.sync_copy(z_hbm.at[pl.ds(base, _BPW)], z_v)
    pltpu.sync_copy(ones_hbm, ones_v)

    @pl.when(sid == 0)
    def _():
        pltpu.sync_copy(zeros_hbm, hist_sh)

    gather.wait()
    plsc.subcore_barrier()

    pltpu.sync_copy(ones_v, hist_sh.at[idx_v], add=True)

    def _rows(r, carry):
        for h in range(_D // 16):
            zv = z_v[r, pl.ds(h * 16, 16)]
            wv = rows_v[r, pl.ds(h * 16, 16)]
            z_v[r, pl.ds(h * 16, 16)] = zv + (wv - zv)
        return carry
    lax.fori_loop(0, _BPW, _rows, 0)

    pltpu.sync_copy(z_v, zq_hbm.at[pl.ds(base, _BPW)])

    plsc.subcore_barrier()

    @pl.when(sid == 0)
    def _():
        pltpu.sync_copy(hist_sh, counts_hbm.at[cid])


def _sc_call(W, idx, z_flat, ones, zeros):
    mesh = plsc.VectorSubcoreMesh(core_axis_name="c", subcore_axis_name="s")
    f = pl.kernel(
        _sc_body,
        out_type=[jax.ShapeDtypeStruct((_B, _D), jnp.float32),
                  jax.ShapeDtypeStruct((_NC, _V), jnp.float32)],
        mesh=mesh,
        scratch_types=[
            pltpu.VMEM((_BPW,), jnp.int32),
            pltpu.VMEM((_BPW, 128), jnp.float32),
            pltpu.VMEM((_BPW, _D), jnp.float32),
            pltpu.VMEM((_BPW,), jnp.float32),
            pltpu.VMEM_SHARED((_V,), jnp.float32),
            pltpu.SemaphoreType.DMA,
        ],
    )
    return f(W, idx, z_flat, ones, zeros)


def _stats_body(z_ref, w_ref, zq_ref, c_ref, loss_ref, perp_ref, md_ref):
    z = z_ref[...]
    w = w_ref[...]
    q = zq_ref[...]
    diff = q - z
    m = jnp.sum(diff * diff) * (1.0 / (_B * _D))
    loss_ref[...] = (m + _BETA * m).reshape(1, 1)

    szsq = jnp.sum(z * z)
    swsq = jnp.sum(w * w)
    sz = jnp.sum(z, axis=0)
    sw = jnp.sum(w, axis=0)
    md_ref[...] = (szsq * (1.0 / _B) + swsq * (1.0 / _V)
                   - 2.0 * jnp.sum(sz * sw) * (1.0 / (_B * _V))).reshape(1, 1)

    cs = c_ref[0:1, :] + c_ref[1:2, :]
    e = cs * (1.0 / _B)
    ent = jnp.sum(e * jnp.log(e + 1e-10))
    perp_ref[...] = jnp.exp(-ent).reshape(1, 1)


def _stats_call(z_flat, W, zq, counts):
    outs = pl.pallas_call(
        _stats_body,
        out_shape=[jax.ShapeDtypeStruct((1, 1), jnp.float32),
                   jax.ShapeDtypeStruct((1, 1), jnp.float32),
                   jax.ShapeDtypeStruct((1, 1), jnp.float32)],
    )(z_flat, W, zq, counts)
    return outs[0][0, 0], outs[1][0, 0], outs[2][0, 0]


def kernel(z, W):
    zp = jnp.transpose(z, (0, 2, 3, 1))
    z_flat = zp.reshape(_B, _D)
    idx, W_pad = _argmin_call(z_flat, W)

    ones = jnp.ones((_BPW,), jnp.float32)
    zeros = jnp.zeros((_V,), jnp.float32)
    zq_flat, counts = _sc_call(W_pad, idx, z_flat, ones, zeros)

    loss, perp, mean_d = _stats_call(z_flat, W, zq_flat, counts)

    z_q_out = jnp.transpose(zq_flat.reshape(zp.shape), (0, 3, 1, 2))
    return (z_q_out, loss, perp, idx, mean_d)


# speedup vs baseline: 1.9333x; 1.0337x over previous
import functools

import jax
import jax.numpy as jnp
from jax import lax
from jax.experimental import pallas as pl
from jax.experimental.pallas import tpu as pltpu
from jax.experimental.pallas import tpu_sc as plsc

_V = 8192
_D = 32
_B = 4096
_BETA = 0.25
_ROWS = 512

_NC = 2
_NS = 16
_NW = _NC * _NS
_BPW = _B // _NW


def _argmin_body(z_ref, w_ref, idx_ref, wpad_ref):
    zt = z_ref[...]
    w = w_ref[...]
    t2 = lax.dot_general(-2.0 * zt, w, (((1,), (1,)), ((), ())),
                         preferred_element_type=jnp.float32)
    zsq = jnp.sum(zt * zt, axis=1, keepdims=True)
    wsq = jnp.sum(w * w, axis=1)
    d = (zsq + wsq[None, :]) + t2
    minval = jnp.min(d, axis=1, keepdims=True)
    js = lax.broadcasted_iota(jnp.int32, d.shape, 1)
    idx = jnp.min(jnp.where(d == minval, js, _V), axis=1)
    idx_ref[0, 0, :] = idx.astype(jnp.int32)

    @pl.when(pl.program_id(0) == 0)
    def _():
        wpad_ref[:, 0:_D] = w


def _argmin_call(z_flat, W):
    nblocks = _B // _ROWS
    out, wpad = pl.pallas_call(
        _argmin_body,
        grid=(nblocks,),
        in_specs=[pl.BlockSpec((_ROWS, _D), lambda i: (i, 0)),
                  pl.BlockSpec((_V, _D), lambda i: (0, 0))],
        out_specs=[pl.BlockSpec((1, 1, _ROWS), lambda i: (i, 0, 0)),
                   pl.BlockSpec((_V, 128), lambda i: (0, 0))],
        out_shape=[jax.ShapeDtypeStruct((nblocks, 1, _ROWS), jnp.int32),
                   jax.ShapeDtypeStruct((_V, 128), jnp.float32)],
    )(z_flat, W)
    return out.reshape(-1), wpad


def _sc_body(w_hbm, idx_hbm, z_hbm, ones_hbm, zeros_hbm,
             zq_hbm, counts_hbm,
             idx_v, rows_v, z_v, ones_v, hist_sh, sem):
    cid = lax.axis_index("c")
    sid = lax.axis_index("s")
    wid = sid * _NC + cid
    base = wid * _BPW

    pltpu.sync_copy(idx_hbm.at[pl.ds(base, _BPW)], idx_v)
    gather = pltpu.async_copy(w_hbm.at[idx_v], rows_v, sem)
    pltpu.sync_copy(z_hbm.at[pl.ds(base, _BPW)], z_v)
    pltpu.sync_copy(ones_hbm, ones_v)

    @pl.when(sid == 0)
    def _():
        pltpu.sync_copy(zeros_hbm, hist_sh)

    gather.wait()
    plsc.subcore_barrier()

    pltpu.sync_copy(ones_v, hist_sh.at[idx_v], add=True)

    def _rows(r, carry):
        for h in range(_D // 16):
            zv = z_v[r, pl.ds(h * 16, 16)]
            wv = rows_v[r, pl.ds(h * 16, 16)]
            z_v[r, pl.ds(h * 16, 16)] = zv + (wv - zv)
        return carry
    lax.fori_loop(0, _BPW, _rows, 0)

    pltpu.sync_copy(z_v, zq_hbm.at[pl.ds(base, _BPW)])

    plsc.subcore_barrier()

    @pl.when(sid == 0)
    def _():
        pltpu.sync_copy(hist_sh, counts_hbm.at[cid])


def _sc_call(W, idx, z_flat, ones, zeros):
    mesh = plsc.VectorSubcoreMesh(core_axis_name="c", subcore_axis_name="s")
    f = pl.kernel(
        _sc_body,
        out_type=[jax.ShapeDtypeStruct((_B, _D), jnp.float32),
                  jax.ShapeDtypeStruct((_NC, _V), jnp.float32)],
        mesh=mesh,
        scratch_types=[
            pltpu.VMEM((_BPW,), jnp.int32),
            pltpu.VMEM((_BPW, 128), jnp.float32),
            pltpu.VMEM((_BPW, _D), jnp.float32),
            pltpu.VMEM((_BPW,), jnp.float32),
            pltpu.VMEM_SHARED((_V,), jnp.float32),
            pltpu.SemaphoreType.DMA,
        ],
    )
    return f(W, idx, z_flat, ones, zeros)


def _stats_body(z_ref, w_ref, zq_ref, c_ref, loss_ref, perp_ref, md_ref):
    z = z_ref[...]
    w = w_ref[...]
    q = zq_ref[...]
    diff = q - z
    m = jnp.sum(diff * diff) * (1.0 / (_B * _D))
    loss_ref[...] = (m + _BETA * m).reshape(1, 1)

    szsq = jnp.sum(z * z)
    swsq = jnp.sum(w * w)
    sz = jnp.sum(z, axis=0)
    sw = jnp.sum(w, axis=0)
    md_ref[...] = (szsq * (1.0 / _B) + swsq * (1.0 / _V)
                   - 2.0 * jnp.sum(sz * sw) * (1.0 / (_B * _V))).reshape(1, 1)

    cs = c_ref[0:1, :] + c_ref[1:2, :]
    e = cs * (1.0 / _B)
    ent = jnp.sum(e * jnp.log(e + 1e-10))
    perp_ref[...] = jnp.exp(-ent).reshape(1, 1)


def _stats_call(z_flat, W, zq, counts):
    outs = pl.pallas_call(
        _stats_body,
        out_shape=[jax.ShapeDtypeStruct((1, 1), jnp.float32),
                   jax.ShapeDtypeStruct((1, 1), jnp.float32),
                   jax.ShapeDtypeStruct((1, 1), jnp.float32)],
    )(z_flat, W, zq, counts)
    return outs[0][0, 0], outs[1][0, 0], outs[2][0, 0]


def kernel(z, W):
    zp = jnp.transpose(z, (0, 2, 3, 1))
    z_flat = zp.reshape(_B, _D)
    idx, W_pad = _argmin_call(z_flat, W)

    ones = jnp.ones((_BPW,), jnp.float32)
    zeros = jnp.zeros((_V,), jnp.float32)
    zq_flat, counts = _sc_call(W_pad, idx, z_flat, ones, zeros)

    loss, perp, mean_d = _stats_call(z_flat, W, zq_flat, counts)

    z_q_out = jnp.transpose(zq_flat.reshape(zp.shape), (0, 3, 1, 2))
    return (z_q_out, loss, perp, idx, mean_d)
